# baseline (speedup 1.0000x reference)
import functools

import jax
import jax.numpy as jnp
from jax import lax
from jax.experimental import pallas as pl
from jax.experimental.pallas import tpu as pltpu
from jax.experimental.pallas import tpu_sc as plsc

_NC, _NS = 2, 16
_NW = _NC * _NS


def _gather3(emb2, kgg2, rel2, pe, oe, pk, ok, pr, orr, B, D):
    b_per_w = B // _NW
    CH = 256
    n_chunk = b_per_w // CH
    mesh = plsc.VectorSubcoreMesh(core_axis_name="c", subcore_axis_name="s")

    @functools.partial(
        pl.kernel,
        mesh=mesh,
        compiler_params=pltpu.CompilerParams(needs_layout_passes=False),
        out_type=(
            jax.ShapeDtypeStruct((D, B), jnp.float32),
            jax.ShapeDtypeStruct((D, B), jnp.float32),
            jax.ShapeDtypeStruct((D, B), jnp.float32),
        ),
        scratch_types=[
            pltpu.VMEM((CH,), jnp.int32),
            pltpu.VMEM((b_per_w,), jnp.int32),
            pltpu.VMEM((CH, 2 * 64), jnp.float32),
            pltpu.VMEM((D, b_per_w), jnp.float32),
            pltpu.SemaphoreType.DMA,
        ],
    )
    def k(emb2_h, kgg2_h, rel2_h, pe_h, oe_h, pk_h, ok_h, pr_h, or_h,
          o1, o2, o3, pid_v, off_v, packed_v, obT_v, sem):
        wid = lax.axis_index("s") * _NC + lax.axis_index("c")
        base = wid * b_per_w
        lanes = lax.iota(jnp.int32, 16)

        def do_table(tab2, p_h, o_h, out_h):
            pltpu.sync_copy(o_h.at[pl.ds(base, b_per_w)], off_v)

            def chunk(ch, carry):
                pltpu.sync_copy(p_h.at[pl.ds(base + ch * CH, CH)], pid_v)
                pltpu.async_copy(tab2.at[pid_v], packed_v, sem).wait()

                def gbody(g, c2):
                    rowv = g * 16 + lanes
                    off16 = off_v[pl.ds(ch * CH + g * 16, 16)]
                    colv = ch * CH + rowv
                    for c in range(D):
                        cc = jnp.full((16,), c, jnp.int32)
                        vals = plsc.load_gather(packed_v, [rowv, off16 + c])
                        plsc.store_scatter(obT_v, [cc, colv], vals)
                    return c2

                lax.fori_loop(0, CH // 16, gbody, 0)
                return carry

            lax.fori_loop(0, n_chunk, chunk, 0)
            pltpu.sync_copy(obT_v, out_h.at[:, pl.ds(base, b_per_w)])

        do_table(emb2_h, pe_h, oe_h, o1)
        do_table(kgg2_h, pk_h, ok_h, o2)
        do_table(rel2_h, pr_h, or_h, o3)

    return k(emb2, kgg2, rel2, pe, oe, pk, ok, pr, orr)


def kernel(embedding, kgg_table, relation_table, scg_ids, relation_ids,
           kgg_ids):
    B, D = scg_ids.shape[0], embedding.shape[1]
    scg_ids = scg_ids.astype(jnp.int32)
    relation_ids = relation_ids.astype(jnp.int32)
    kgg_ids = kgg_ids.astype(jnp.int32)
    emb2 = embedding.reshape(-1, 2 * D)
    kgg2 = kgg_table.reshape(-1, 2 * D)
    rel2 = relation_table.reshape(-1, 2 * D)
    o1, o2, o3 = _gather3(
        emb2, kgg2, rel2,
        scg_ids >> 1, (scg_ids & 1) << 6,
        kgg_ids >> 1, (kgg_ids & 1) << 6,
        relation_ids >> 1, (relation_ids & 1) << 6,
        B, D)
    return (o1.T, o2.T, o3.T)


# speedup vs baseline: 1.0607x; 1.0607x over previous
import functools

import jax
import jax.numpy as jnp
from jax import lax
from jax.experimental import pallas as pl
from jax.experimental.pallas import tpu as pltpu
from jax.experimental.pallas import tpu_sc as plsc

_NS = 16


def _gather3(embedding, kgg_table, relation_table, scg_ids, relation_ids,
             kgg_ids):
    B = scg_ids.shape[0]
    D = embedding.shape[1]
    b_per_w = B // _NS
    CH = 512
    n_chunk = b_per_w // CH
    mesh = plsc.VectorSubcoreMesh(core_axis_name="c", subcore_axis_name="s",
                                  num_cores=1)

    @functools.partial(
        pl.kernel,
        mesh=mesh,
        compiler_params=pltpu.CompilerParams(use_tc_tiling_on_sc=False),
        out_type=(
            jax.ShapeDtypeStruct((B, D), jnp.float32),
            jax.ShapeDtypeStruct((B, D), jnp.float32),
            jax.ShapeDtypeStruct((B, D), jnp.float32),
        ),
        scratch_types=[
            pltpu.VMEM((CH,), jnp.int32),
            pltpu.VMEM((CH, 64), jnp.float32),
            pltpu.VMEM((CH, 64), jnp.float32),
            pltpu.SemaphoreType.DMA,
            pltpu.SemaphoreType.DMA,
            pltpu.SemaphoreType.DMA,
        ],
    )
    def k(emb_h, kgg_h, rel_h, scg_h, relid_h, kggid_h,
          out_scg, out_kgg, out_rel,
          idx_v, rows_a, rows_b, sem_a, sem_b, sem_o):
        wid = lax.axis_index("s")
        base = wid * b_per_w

        def do_chunk(tab_h, ids_h, out_h, cbase, rows_v, sem):
            pltpu.sync_copy(ids_h.at[pl.ds(cbase, CH)], idx_v)
            pltpu.async_copy(tab_h.at[idx_v], rows_v, sem).wait()
            pltpu.sync_copy(rows_v, out_h.at[pl.ds(cbase, CH)])

        for c in range(n_chunk):
            do_chunk(emb_h, scg_h, out_scg, base + c * CH, rows_a, sem_a)
            do_chunk(kgg_h, kggid_h, out_kgg, base + c * CH, rows_b, sem_b)
        for c in range(n_chunk):
            do_chunk(rel_h, relid_h, out_rel, base + c * CH, rows_a, sem_a)

    return k(embedding, kgg_table, relation_table, scg_ids, relation_ids,
             kgg_ids)


def kernel(embedding, kgg_table, relation_table, scg_ids, relation_ids,
           kgg_ids):
    return _gather3(embedding, kgg_table, relation_table,
                    scg_ids.astype(jnp.int32), relation_ids.astype(jnp.int32),
                    kgg_ids.astype(jnp.int32))


# speedup vs baseline: 1.0731x; 1.0117x over previous
import functools

import jax
import jax.numpy as jnp
from jax import lax
from jax.experimental import pallas as pl
from jax.experimental.pallas import tpu as pltpu
from jax.experimental.pallas import tpu_sc as plsc


def _gather3(embedding, kgg_table, relation_table, scg_ids, relation_ids,
             kgg_ids):
    B = scg_ids.shape[0]
    D = embedding.shape[1]
    NC, NS = 2, 16
    NW = NC * NS
    b_per_w = B // NW
    mesh = plsc.VectorSubcoreMesh(core_axis_name="c", subcore_axis_name="s")

    @functools.partial(
        pl.kernel,
        mesh=mesh,
        compiler_params=pltpu.CompilerParams(use_tc_tiling_on_sc=False),
        out_type=(
            jax.ShapeDtypeStruct((B, D), jnp.float32),
            jax.ShapeDtypeStruct((B, D), jnp.float32),
            jax.ShapeDtypeStruct((B, D), jnp.float32),
        ),
        scratch_types=[
            pltpu.VMEM((b_per_w,), jnp.int32),
            pltpu.VMEM((b_per_w,), jnp.int32),
            pltpu.VMEM((b_per_w,), jnp.int32),
            pltpu.VMEM((b_per_w, D), jnp.float32),
            pltpu.VMEM((b_per_w, D), jnp.float32),
            pltpu.VMEM((b_per_w, D), jnp.float32),
            pltpu.SemaphoreType.DMA,
            pltpu.SemaphoreType.DMA,
            pltpu.SemaphoreType.DMA,
            pltpu.SemaphoreType.DMA,
        ],
    )
    def k(emb_hbm, kgg_hbm, rel_hbm, scg_hbm, relid_hbm, kggid_hbm,
          out_scg, out_kgg, out_rel,
          idx_scg, idx_kgg, idx_rel, rows_scg, rows_kgg, rows_rel,
          sem_scg, sem_kgg, sem_rel, sem_out):
        wid = lax.axis_index("s") * NC + lax.axis_index("c")
        base = wid * b_per_w
        pltpu.sync_copy(scg_hbm.at[pl.ds(base, b_per_w)], idx_scg)
        pltpu.sync_copy(kggid_hbm.at[pl.ds(base, b_per_w)], idx_kgg)
        pltpu.sync_copy(relid_hbm.at[pl.ds(base, b_per_w)], idx_rel)
        g1 = pltpu.async_copy(emb_hbm.at[idx_scg], rows_scg, sem_scg)
        g2 = pltpu.async_copy(kgg_hbm.at[idx_kgg], rows_kgg, sem_kgg)
        g3 = pltpu.async_copy(rel_hbm.at[idx_rel], rows_rel, sem_rel)
        g1.wait()
        w1 = pltpu.async_copy(rows_scg, out_scg.at[pl.ds(base, b_per_w)],
                              sem_out)
        g2.wait()
        w2 = pltpu.async_copy(rows_kgg, out_kgg.at[pl.ds(base, b_per_w)],
                              sem_out)
        g3.wait()
        w3 = pltpu.async_copy(rows_rel, out_rel.at[pl.ds(base, b_per_w)],
                              sem_out)
        w1.wait()
        w2.wait()
        w3.wait()

    return k(embedding, kgg_table, relation_table, scg_ids, relation_ids,
             kgg_ids)


def kernel(embedding, kgg_table, relation_table, scg_ids, relation_ids,
           kgg_ids):
    return _gather3(embedding, kgg_table, relation_table,
                    scg_ids.astype(jnp.int32), relation_ids.astype(jnp.int32),
                    kgg_ids.astype(jnp.int32))


# speedup vs baseline: 1.0805x; 1.0069x over previous
import functools

import jax
import jax.numpy as jnp
from jax import lax
from jax.experimental import pallas as pl
from jax.experimental.pallas import tpu as pltpu
from jax.experimental.pallas import tpu_sc as plsc

_NC, _NS = 2, 16
_NW = _NC * _NS


def _gather1(table, ids):
    B = ids.shape[0]
    D = table.shape[1]
    b_per_w = B // _NW
    mesh = plsc.VectorSubcoreMesh(core_axis_name="c", subcore_axis_name="s")

    @functools.partial(
        pl.kernel,
        mesh=mesh,
        compiler_params=pltpu.CompilerParams(use_tc_tiling_on_sc=False),
        out_type=jax.ShapeDtypeStruct((B, D), jnp.float32),
        scratch_types=[
            pltpu.VMEM((b_per_w,), jnp.int32),
            pltpu.VMEM((b_per_w, D), jnp.float32),
            pltpu.SemaphoreType.DMA,
        ],
    )
    def k(tab_h, ids_h, out_h, idx_v, rows_v, sem):
        wid = lax.axis_index("s") * _NC + lax.axis_index("c")
        base = wid * b_per_w
        pltpu.sync_copy(ids_h.at[pl.ds(base, b_per_w)], idx_v)
        pltpu.async_copy(tab_h.at[idx_v], rows_v, sem).wait()
        pltpu.sync_copy(rows_v, out_h.at[pl.ds(base, b_per_w)])

    return k(table, ids)


def kernel(embedding, kgg_table, relation_table, scg_ids, relation_ids,
           kgg_ids):
    return (
        _gather1(embedding, scg_ids.astype(jnp.int32)),
        _gather1(kgg_table, kgg_ids.astype(jnp.int32)),
        _gather1(relation_table, relation_ids.astype(jnp.int32)),
    )
